# Initial kernel scaffold; baseline (speedup 1.0000x reference)
import jax
import jax.numpy as jnp
from jax.experimental import pallas as pl


def kernel(feat, segment_ids, last_nodes, W_u, W_v, b_v, W_e, prelu_w):
    raise NotImplementedError("write your pallas kernel here")


# speedup vs baseline: 3.5314x; 3.5314x over previous
import functools

import jax
import jax.numpy as jnp
from jax import lax
from jax.experimental import pallas as pl
from jax.experimental.pallas import tpu as pltpu
from jax.experimental.pallas import tpu_sc as plsc

BLK = 512


def _sc_gather(feat, idx32):
    b = idx32.shape[0]
    d = feat.shape[1]
    window = 128
    mesh = plsc.VectorSubcoreMesh(core_axis_name="core", subcore_axis_name="subcore")
    indices = idx32.reshape((1, b))

    @functools.partial(
        pl.kernel,
        out_type=jax.ShapeDtypeStruct((b, d), feat.dtype),
        mesh=mesh,
    )
    def kern(x_hbm, i_hbm, o_hbm):
        def body(i_vmem, o_vmem):
            pltpu.sync_copy(x_hbm.at[i_vmem.at[0]], o_vmem)

        pltpu.emit_pipeline(
            body,
            grid=(b // window,),
            in_specs=[pl.BlockSpec((1, window), index_map=lambda i: (0, i))],
            out_specs=[pl.BlockSpec((window, d), index_map=lambda i: (i, 0))],
            core_axis_name="subcore",
            dimension_semantics=(pltpu.PARALLEL,),
        )(i_hbm, o_hbm)

    return kern(feat, indices)


def _tc_body(feat_ref, seg_ref, gat_ref, wu_ref, wv_ref, bv_ref, we_ref, pw_ref,
             out_ref, fv_s, eexp_s, den_s, *, n, b, nb):
    p = pl.program_id(0)
    i = pl.program_id(1)
    f32 = jnp.float32

    @pl.when(jnp.logical_and(p == 0, i == 0))
    def _prologue():
        fv = lax.dot_general(gat_ref[...], wv_ref[...], (((1,), (1,)), ((), ())),
                             preferred_element_type=f32)
        fv_s[...] = fv + bv_ref[...]
        den_s[...] = jnp.zeros_like(den_s)

    seg = seg_ref[...]
    onehot = (seg == lax.broadcasted_iota(jnp.int32, (BLK, b), 1)).astype(f32)
    valid_row = (i * BLK + lax.broadcasted_iota(jnp.int32, (1, BLK), 1)) < n

    @pl.when(p == 0)
    def _phase0():
        u = lax.dot_general(feat_ref[...], wu_ref[...], (((1,), (1,)), ((), ())),
                            preferred_element_type=f32)
        fvb = lax.dot_general(onehot, fv_s[...], (((1,), (0,)), ((), ())),
                              preferred_element_type=f32)
        s = jax.nn.sigmoid(u + fvb)
        e_row = lax.dot_general(we_ref[...], s, (((1,), (1,)), ((), ())),
                                preferred_element_type=f32)
        eexp = jnp.where(valid_row, jnp.exp(e_row), 0.0)
        eexp_s[pl.ds(i, 1), :] = eexp
        den_s[...] += lax.dot_general(eexp, onehot, (((1,), (0,)), ((), ())),
                                      preferred_element_type=f32)

    @pl.when(p == 1)
    def _phase1():
        eexp = eexp_s[pl.ds(i, 1), :]
        dg = lax.dot_general(den_s[...], onehot, (((1,), (1,)), ((), ())),
                             preferred_element_type=f32)
        alpha_col = jnp.transpose(eexp / dg, (1, 0))
        valid_col = (i * BLK + lax.broadcasted_iota(jnp.int32, (BLK, 1), 0)) < n
        featn = jnp.where(valid_col, feat_ref[...] * alpha_col, 0.0)
        contrib = lax.dot_general(onehot.astype(jnp.bfloat16),
                                  featn.astype(jnp.bfloat16),
                                  (((0,), (0,)), ((), ())),
                                  preferred_element_type=f32)

        @pl.when(i == 0)
        def _():
            out_ref[...] = contrib

        @pl.when(i > 0)
        def _():
            out_ref[...] += contrib

        @pl.when(i == nb - 1)
        def _():
            acc = out_ref[...]
            out_ref[...] = jnp.where(acc > 0, acc, pw_ref[...] * acc)


def _tc_main(feat, seg_pad, gathered, W_u, W_v, b_v, W_e, prelu_w):
    n, d = feat.shape
    h = W_u.shape[0]
    b = gathered.shape[0]
    nb = seg_pad.shape[0] // BLK
    nb_pad = ((nb + 7) // 8) * 8

    grid = (2, nb)
    body = functools.partial(_tc_body, n=n, b=b, nb=nb)
    return pl.pallas_call(
        body,
        grid=grid,
        in_specs=[
            pl.BlockSpec((BLK, d), lambda p, i: (i, 0)),
            pl.BlockSpec((BLK, 1), lambda p, i: (i, 0)),
            pl.BlockSpec((b, d), lambda p, i: (0, 0)),
            pl.BlockSpec((h, d), lambda p, i: (0, 0)),
            pl.BlockSpec((h, d), lambda p, i: (0, 0)),
            pl.BlockSpec((1, h), lambda p, i: (0, 0)),
            pl.BlockSpec((1, h), lambda p, i: (0, 0)),
            pl.BlockSpec((1, d), lambda p, i: (0, 0)),
        ],
        out_specs=pl.BlockSpec((b, d), lambda p, i: (0, 0)),
        out_shape=jax.ShapeDtypeStruct((b, d), jnp.float32),
        scratch_shapes=[
            pltpu.VMEM((b, h), jnp.float32),
            pltpu.VMEM((nb_pad, BLK), jnp.float32),
            pltpu.VMEM((1, b), jnp.float32),
        ],
        compiler_params=pltpu.CompilerParams(
            dimension_semantics=("arbitrary", "arbitrary"),
        ),
    )(feat, seg_pad, gathered, W_u, W_v, b_v, W_e, prelu_w)


def kernel(feat, segment_ids, last_nodes, W_u, W_v, b_v, W_e, prelu_w):
    n, d = feat.shape
    h = W_u.shape[0]
    b = last_nodes.shape[0]
    nb = -(-n // BLK)
    np_ = nb * BLK

    seg32 = segment_ids.astype(jnp.int32)
    seg_pad = jnp.full((np_,), b, jnp.int32).at[:n].set(seg32).reshape(np_, 1)
    idx32 = last_nodes.astype(jnp.int32)

    gathered = _sc_gather(feat, idx32)
    return _tc_main(feat, seg_pad, gathered,
                    W_u, W_v,
                    b_v.reshape(1, h).astype(jnp.float32),
                    W_e, prelu_w.reshape(1, d).astype(jnp.float32))
